# Initial kernel scaffold; baseline (speedup 1.0000x reference)
import jax
import jax.numpy as jnp
from jax.experimental import pallas as pl


def kernel(charges, neighbor_indices, neighbor_distances):
    raise NotImplementedError("write your pallas kernel here")


# speedup vs baseline: 15.2114x; 15.2114x over previous
import functools

import jax
import jax.numpy as jnp
from jax import lax
from jax.experimental import pallas as pl
from jax.experimental.pallas import tpu as pltpu
from jax.experimental.pallas import tpu_sc as plsc

N_NODES = 100000
NPAD = 100096
N_EDGES = 6400000
NC = 2
NS = 16
NW = NC * NS
EW = N_EDGES // NW
K = 2000
G = 80
NG = K // G
R = EW // K
L = 16
ZCHUNK = NPAD // NS

_mesh = plsc.VectorSubcoreMesh(
    core_axis_name="c", subcore_axis_name="s", num_cores=NC, num_subcores=NS
)


@functools.partial(
    pl.kernel,
    out_type=jax.ShapeDtypeStruct((NC * NPAD,), jnp.float32),
    mesh=_mesh,
    compiler_params=pltpu.CompilerParams(needs_layout_passes=False),
    scratch_types=[
        pltpu.VMEM((NPAD,), jnp.float32),
        pltpu.VMEM((2 * K,), jnp.int32),
        pltpu.VMEM((K,), jnp.float32),
        pltpu.VMEM((NG, G), jnp.float32),
        pltpu.VMEM((NG, G), jnp.float32),
        pltpu.VMEM((NG, G), jnp.int32),
        pltpu.VMEM((NG, G), jnp.int32),
        pltpu.VMEM_SHARED((NPAD,), jnp.float32),
    ],
)
def _sc_potential(charges_hbm, idx_hbm, dist_hbm, out_hbm,
                  table_v, idxc_v, dist_v, vi_v, vj_v, si_v, sj_v,
                  acc_sh):
    c = lax.axis_index("c")
    s = lax.axis_index("s")
    w = c * NS + s

    zeros = jnp.zeros((L,), jnp.float32)

    def _zfill(t, carry):
        table_v[pl.ds(t * L, L)] = zeros
        return carry

    lax.fori_loop(0, ZCHUNK // L, _zfill, 0)
    pltpu.sync_copy(table_v.at[pl.ds(0, ZCHUNK)],
                    acc_sh.at[pl.ds(s * ZCHUNK, ZCHUNK)])

    pltpu.sync_copy(charges_hbm, table_v)
    plsc.subcore_barrier()

    lanes = lax.broadcasted_iota(jnp.int32, (L,), 0)
    ebase = w * EW

    def _round(r, carry):
        base = ebase + r * K
        pltpu.sync_copy(idx_hbm.at[pl.ds(2 * base, 2 * K)], idxc_v)
        pltpu.sync_copy(dist_hbm.at[pl.ds(base, K)], dist_v)

        def _group(g, carry2):
            for u in range(G // L):
                e0 = g * G + u * L
                pos = 2 * e0 + 2 * lanes
                ii = plsc.load_gather(idxc_v, [pos])
                jj = plsc.load_gather(idxc_v, [pos + 1])
                dd = dist_v[pl.ds(e0, L)]
                ci = plsc.load_gather(table_v, [ii])
                cj = plsc.load_gather(table_v, [jj])
                p = 0.5 / dd
                vi_v[g, pl.ds(u * L, L)] = cj * p
                vj_v[g, pl.ds(u * L, L)] = ci * p
                si_v[g, pl.ds(u * L, L)] = ii
                sj_v[g, pl.ds(u * L, L)] = jj
            pltpu.sync_copy(vi_v.at[g], acc_sh.at[si_v.at[g]], add=True)
            pltpu.sync_copy(vj_v.at[g], acc_sh.at[sj_v.at[g]], add=True)
            return carry2

        lax.fori_loop(0, NG, _group, 0)
        return carry

    lax.fori_loop(0, R, _round, 0)

    plsc.subcore_barrier()
    pltpu.sync_copy(acc_sh.at[pl.ds(s * ZCHUNK, ZCHUNK)],
                    table_v.at[pl.ds(0, ZCHUNK)])
    pltpu.sync_copy(table_v.at[pl.ds(0, ZCHUNK)],
                    out_hbm.at[pl.ds(c * NPAD + s * ZCHUNK, ZCHUNK)])


def kernel(charges, neighbor_indices, neighbor_distances):
    ch = jnp.pad(charges[:, 0], (0, NPAD - N_NODES))
    idx = neighbor_indices.astype(jnp.int32).reshape(-1)
    parts = _sc_potential(ch, idx, neighbor_distances)
    return (parts[:N_NODES] + parts[NPAD:NPAD + N_NODES]).reshape(N_NODES, 1)


# speedup vs baseline: 156.6348x; 10.2972x over previous
import functools

import jax
import jax.numpy as jnp
from jax import lax
from jax.experimental import pallas as pl
from jax.experimental.pallas import tpu as pltpu
from jax.experimental.pallas import tpu_sc as plsc

N_NODES = 100000
NPAD = 100096
N_EDGES = 6400000
NC = 2
NS = 16
NW = NC * NS
EW = N_EDGES // NW
K = 2000
G = 80
NG = K // G
R = EW // K
L = 16
ZCHUNK = NPAD // NS

_mesh = plsc.VectorSubcoreMesh(
    core_axis_name="c", subcore_axis_name="s", num_cores=NC, num_subcores=NS
)


@functools.partial(
    pl.kernel,
    out_type=jax.ShapeDtypeStruct((NC * NPAD,), jnp.float32),
    mesh=_mesh,
    compiler_params=pltpu.CompilerParams(needs_layout_passes=False),
    scratch_types=[
        pltpu.VMEM((NPAD,), jnp.float32),
        pltpu.VMEM((K,), jnp.int32),
        pltpu.VMEM((K,), jnp.int32),
        pltpu.VMEM((K,), jnp.float32),
        pltpu.VMEM((NG, G), jnp.float32),
        pltpu.VMEM((NG, G), jnp.float32),
        pltpu.VMEM((NG, G), jnp.int32),
        pltpu.VMEM((NG, G), jnp.int32),
        pltpu.VMEM_SHARED((NPAD,), jnp.float32),
    ],
)
def _sc_potential(charges_hbm, ai_hbm, aj_hbm, dist_hbm, out_hbm,
                  table_v, ai_v, aj_v, dist_v, vi_v, vj_v, si_v, sj_v,
                  acc_sh):
    c = lax.axis_index("c")
    s = lax.axis_index("s")
    w = c * NS + s

    zeros = jnp.zeros((L,), jnp.float32)

    def _zfill(t, carry):
        table_v[pl.ds(t * L, L)] = zeros
        return carry

    lax.fori_loop(0, ZCHUNK // L, _zfill, 0)
    pltpu.sync_copy(table_v.at[pl.ds(0, ZCHUNK)],
                    acc_sh.at[pl.ds(s * ZCHUNK, ZCHUNK)])

    pltpu.sync_copy(charges_hbm, table_v)
    plsc.subcore_barrier()

    ebase = w * EW

    def _round(r, carry):
        base = ebase + r * K
        pltpu.sync_copy(ai_hbm.at[pl.ds(base, K)], ai_v)
        pltpu.sync_copy(aj_hbm.at[pl.ds(base, K)], aj_v)
        pltpu.sync_copy(dist_hbm.at[pl.ds(base, K)], dist_v)

        def _group(g, carry2):
            for u in range(G // L):
                e0 = g * G + u * L
                ii = ai_v[pl.ds(e0, L)]
                jj = aj_v[pl.ds(e0, L)]
                dd = dist_v[pl.ds(e0, L)]
                ci = plsc.load_gather(table_v, [ii])
                cj = plsc.load_gather(table_v, [jj])
                p = 0.5 / dd
                vi_v[g, pl.ds(u * L, L)] = cj * p
                vj_v[g, pl.ds(u * L, L)] = ci * p
                si_v[g, pl.ds(u * L, L)] = ii
                sj_v[g, pl.ds(u * L, L)] = jj
            pltpu.sync_copy(vi_v.at[g], acc_sh.at[si_v.at[g]], add=True)
            pltpu.sync_copy(vj_v.at[g], acc_sh.at[sj_v.at[g]], add=True)
            return carry2

        lax.fori_loop(0, NG, _group, 0)
        return carry

    lax.fori_loop(0, R, _round, 0)

    plsc.subcore_barrier()
    pltpu.sync_copy(acc_sh.at[pl.ds(s * ZCHUNK, ZCHUNK)],
                    table_v.at[pl.ds(0, ZCHUNK)])
    pltpu.sync_copy(table_v.at[pl.ds(0, ZCHUNK)],
                    out_hbm.at[pl.ds(c * NPAD + s * ZCHUNK, ZCHUNK)])


def kernel(charges, neighbor_indices, neighbor_distances):
    ch = jnp.pad(charges[:, 0], (0, NPAD - N_NODES))
    idx = neighbor_indices.astype(jnp.int32)
    parts = _sc_potential(ch, idx[:, 0], idx[:, 1], neighbor_distances)
    return (parts[:N_NODES] + parts[NPAD:NPAD + N_NODES]).reshape(N_NODES, 1)


# speedup vs baseline: 451.6267x; 2.8833x over previous
import functools

import jax
import jax.numpy as jnp
from jax import lax
from jax.experimental import pallas as pl
from jax.experimental.pallas import tpu as pltpu
from jax.experimental.pallas import tpu_sc as plsc

N_NODES = 100000
NPAD = 100096
N_EDGES = 6400000
NC = 2
NS = 16
NW = NC * NS
EW = N_EDGES // NW
K = 800
G = 80
NG = K // G
R = EW // K
L = 16
ZCHUNK = NPAD // NS

_mesh = plsc.VectorSubcoreMesh(
    core_axis_name="c", subcore_axis_name="s", num_cores=NC, num_subcores=NS
)


@functools.partial(
    pl.kernel,
    out_type=jax.ShapeDtypeStruct((NC * NPAD,), jnp.float32),
    mesh=_mesh,
    compiler_params=pltpu.CompilerParams(needs_layout_passes=False),
    scratch_types=[
        pltpu.VMEM((NPAD,), jnp.float32),
        pltpu.VMEM((K,), jnp.int32),
        pltpu.VMEM((K,), jnp.int32),
        pltpu.VMEM((K,), jnp.float32),
        pltpu.VMEM((K,), jnp.int32),
        pltpu.VMEM((K,), jnp.int32),
        pltpu.VMEM((K,), jnp.float32),
        pltpu.VMEM((NG, G), jnp.float32),
        pltpu.VMEM((NG, G), jnp.float32),
        pltpu.VMEM((NG, G), jnp.int32),
        pltpu.VMEM((NG, G), jnp.int32),
        pltpu.VMEM_SHARED((NPAD,), jnp.float32),
        pltpu.SemaphoreType.DMA,
        pltpu.SemaphoreType.DMA,
        pltpu.SemaphoreType.DMA,
    ],
)
def _sc_potential(charges_hbm, ai_hbm, aj_hbm, dist_hbm, out_hbm,
                  table_v, aiA, ajA, ddA, aiB, ajB, ddB,
                  vi_v, vj_v, si_v, sj_v, acc_sh,
                  semA, semB, sc_sem):
    c = lax.axis_index("c")
    s = lax.axis_index("s")
    w = c * NS + s

    zeros = jnp.zeros((L,), jnp.float32)

    def _zfill(t, carry):
        table_v[pl.ds(t * L, L)] = zeros
        return carry

    lax.fori_loop(0, ZCHUNK // L, _zfill, 0)
    pltpu.sync_copy(table_v.at[pl.ds(0, ZCHUNK)],
                    acc_sh.at[pl.ds(s * ZCHUNK, ZCHUNK)])

    pltpu.sync_copy(charges_hbm, table_v)
    plsc.subcore_barrier()

    ebase = w * EW

    def _fetch(r, ai_b, aj_b, dd_b, sem):
        base = ebase + r * K
        pltpu.async_copy(ai_hbm.at[pl.ds(base, K)], ai_b, sem)
        pltpu.async_copy(aj_hbm.at[pl.ds(base, K)], aj_b, sem)
        pltpu.async_copy(dist_hbm.at[pl.ds(base, K)], dd_b, sem)

    def _wait_inputs(ai_b, aj_b, dd_b, sem):
        pltpu.make_async_copy(ai_hbm.at[pl.ds(0, K)], ai_b, sem).wait()
        pltpu.make_async_copy(aj_hbm.at[pl.ds(0, K)], aj_b, sem).wait()
        pltpu.make_async_copy(dist_hbm.at[pl.ds(0, K)], dd_b, sem).wait()

    def _drain_scatter(n):
        for _ in range(n):
            pltpu.make_async_copy(vi_v.at[0], acc_sh.at[si_v.at[0]],
                                  sc_sem).wait()
            pltpu.make_async_copy(vj_v.at[0], acc_sh.at[sj_v.at[0]],
                                  sc_sem).wait()

    def _process(ai_b, aj_b, dd_b):
        def _group(g, carry):
            for u in range(G // L):
                e0 = g * G + u * L
                ii = ai_b[pl.ds(e0, L)]
                jj = aj_b[pl.ds(e0, L)]
                dd = dd_b[pl.ds(e0, L)]
                ci = plsc.load_gather(table_v, [ii])
                cj = plsc.load_gather(table_v, [jj])
                p = 0.5 / dd
                vi_v[g, pl.ds(u * L, L)] = cj * p
                vj_v[g, pl.ds(u * L, L)] = ci * p
                si_v[g, pl.ds(u * L, L)] = ii
                sj_v[g, pl.ds(u * L, L)] = jj
            pltpu.async_copy(vi_v.at[g], acc_sh.at[si_v.at[g]], sc_sem,
                             add=True)
            pltpu.async_copy(vj_v.at[g], acc_sh.at[sj_v.at[g]], sc_sem,
                             add=True)

            @pl.when(g >= 2)
            def _():
                _drain_scatter(1)

            return carry

        lax.fori_loop(0, NG, _group, 0)
        _drain_scatter(2)

    _fetch(0, aiA, ajA, ddA, semA)

    def _pair(t, carry):
        r = 2 * t
        _wait_inputs(aiA, ajA, ddA, semA)
        _fetch(r + 1, aiB, ajB, ddB, semB)
        _process(aiA, ajA, ddA)
        _wait_inputs(aiB, ajB, ddB, semB)

        @pl.when(r + 2 < R)
        def _():
            _fetch(r + 2, aiA, ajA, ddA, semA)

        _process(aiB, ajB, ddB)
        return carry

    lax.fori_loop(0, R // 2, _pair, 0)

    plsc.subcore_barrier()
    pltpu.sync_copy(acc_sh.at[pl.ds(s * ZCHUNK, ZCHUNK)],
                    table_v.at[pl.ds(0, ZCHUNK)])
    pltpu.sync_copy(table_v.at[pl.ds(0, ZCHUNK)],
                    out_hbm.at[pl.ds(c * NPAD + s * ZCHUNK, ZCHUNK)])


def kernel(charges, neighbor_indices, neighbor_distances):
    ch = jnp.pad(charges[:, 0], (0, NPAD - N_NODES))
    idx = neighbor_indices.astype(jnp.int32)
    parts = _sc_potential(ch, idx[:, 0], idx[:, 1], neighbor_distances)
    return (parts[:N_NODES] + parts[NPAD:NPAD + N_NODES]).reshape(N_NODES, 1)


# speedup vs baseline: 480.3309x; 1.0636x over previous
import functools

import jax
import jax.numpy as jnp
from jax import lax
from jax.experimental import pallas as pl
from jax.experimental.pallas import tpu as pltpu
from jax.experimental.pallas import tpu_sc as plsc

N_NODES = 100000
NPAD = 100096
N_EDGES = 6400000
NC = 2
NS = 16
NW = NC * NS
EW = N_EDGES // NW
K = 800
G = 80
NG = K // G
R = EW // K
L = 16
ZCHUNK = NPAD // NS

_mesh = plsc.VectorSubcoreMesh(
    core_axis_name="c", subcore_axis_name="s", num_cores=NC, num_subcores=NS
)


@functools.partial(
    pl.kernel,
    out_type=jax.ShapeDtypeStruct((NC * NPAD,), jnp.float32),
    mesh=_mesh,
    compiler_params=pltpu.CompilerParams(needs_layout_passes=False),
    scratch_types=[
        pltpu.VMEM((NPAD,), jnp.float32),
        pltpu.VMEM((K,), jnp.int32),
        pltpu.VMEM((K,), jnp.int32),
        pltpu.VMEM((K,), jnp.float32),
        pltpu.VMEM((K,), jnp.int32),
        pltpu.VMEM((K,), jnp.int32),
        pltpu.VMEM((K,), jnp.float32),
        pltpu.VMEM((NG, G), jnp.float32),
        pltpu.VMEM((NG, G), jnp.float32),
        pltpu.VMEM((NG, G), jnp.int32),
        pltpu.VMEM((NG, G), jnp.int32),
        pltpu.VMEM_SHARED((NPAD,), jnp.float32),
        pltpu.SemaphoreType.DMA,
        pltpu.SemaphoreType.DMA,
        pltpu.SemaphoreType.DMA,
    ],
)
def _sc_potential(charges_hbm, ai_hbm, aj_hbm, dist_hbm, out_hbm,
                  table_v, aiA, ajA, ddA, aiB, ajB, ddB,
                  vi_v, vj_v, si_v, sj_v, acc_sh,
                  semA, semB, sc_sem):
    c = lax.axis_index("c")
    s = lax.axis_index("s")
    w = c * NS + s

    zeros = jnp.zeros((L,), jnp.float32)

    def _zfill(t, carry):
        table_v[pl.ds(t * L, L)] = zeros
        return carry

    lax.fori_loop(0, ZCHUNK // L, _zfill, 0)
    pltpu.sync_copy(table_v.at[pl.ds(0, ZCHUNK)],
                    acc_sh.at[pl.ds(s * ZCHUNK, ZCHUNK)])

    pltpu.sync_copy(charges_hbm, table_v)
    plsc.subcore_barrier()

    ebase = w * EW

    def _fetch(r, ai_b, aj_b, dd_b, sem):
        base = ebase + r * K
        pltpu.async_copy(ai_hbm.at[pl.ds(base, K)], ai_b, sem)
        pltpu.async_copy(aj_hbm.at[pl.ds(base, K)], aj_b, sem)
        pltpu.async_copy(dist_hbm.at[pl.ds(base, K)], dd_b, sem)

    def _wait_inputs(ai_b, aj_b, dd_b, sem):
        pltpu.make_async_copy(ai_hbm.at[pl.ds(0, K)], ai_b, sem).wait()
        pltpu.make_async_copy(aj_hbm.at[pl.ds(0, K)], aj_b, sem).wait()
        pltpu.make_async_copy(dist_hbm.at[pl.ds(0, K)], dd_b, sem).wait()

    def _drain_scatter(n):
        for _ in range(n):
            pltpu.make_async_copy(vi_v.at[0], acc_sh.at[si_v.at[0]],
                                  sc_sem).wait()
            pltpu.make_async_copy(vj_v.at[0], acc_sh.at[sj_v.at[0]],
                                  sc_sem).wait()

    def _process(ai_b, aj_b, dd_b):
        def _group(g, carry):
            for u in range(G // L):
                e0 = g * G + u * L
                ii = ai_b[pl.ds(e0, L)]
                jj = aj_b[pl.ds(e0, L)]
                dd = dd_b[pl.ds(e0, L)]
                ci = plsc.load_gather(table_v, [ii])
                cj = plsc.load_gather(table_v, [jj])
                p = 0.5 / dd
                vi_v[g, pl.ds(u * L, L)] = cj * p
                vj_v[g, pl.ds(u * L, L)] = ci * p
                si_v[g, pl.ds(u * L, L)] = ii
                sj_v[g, pl.ds(u * L, L)] = jj
            pltpu.async_copy(vi_v.at[g], acc_sh.at[si_v.at[g]], sc_sem,
                             add=True)
            pltpu.async_copy(vj_v.at[g], acc_sh.at[sj_v.at[g]], sc_sem,
                             add=True)
            return carry

        lax.fori_loop(0, NG, _group, 0)
        _drain_scatter(NG)

    _fetch(0, aiA, ajA, ddA, semA)

    def _pair(t, carry):
        r = 2 * t
        _wait_inputs(aiA, ajA, ddA, semA)
        _fetch(r + 1, aiB, ajB, ddB, semB)
        _process(aiA, ajA, ddA)
        _wait_inputs(aiB, ajB, ddB, semB)

        @pl.when(r + 2 < R)
        def _():
            _fetch(r + 2, aiA, ajA, ddA, semA)

        _process(aiB, ajB, ddB)
        return carry

    lax.fori_loop(0, R // 2, _pair, 0)

    plsc.subcore_barrier()
    pltpu.sync_copy(acc_sh.at[pl.ds(s * ZCHUNK, ZCHUNK)],
                    table_v.at[pl.ds(0, ZCHUNK)])
    pltpu.sync_copy(table_v.at[pl.ds(0, ZCHUNK)],
                    out_hbm.at[pl.ds(c * NPAD + s * ZCHUNK, ZCHUNK)])


def kernel(charges, neighbor_indices, neighbor_distances):
    ch = jnp.pad(charges[:, 0], (0, NPAD - N_NODES))
    idx = neighbor_indices.astype(jnp.int32)
    parts = _sc_potential(ch, idx[:, 0], idx[:, 1], neighbor_distances)
    return (parts[:N_NODES] + parts[NPAD:NPAD + N_NODES]).reshape(N_NODES, 1)


# speedup vs baseline: 507.5637x; 1.0567x over previous
import functools

import jax
import jax.numpy as jnp
from jax import lax
from jax.experimental import pallas as pl
from jax.experimental.pallas import tpu as pltpu
from jax.experimental.pallas import tpu_sc as plsc

N_NODES = 100000
NPAD = 100096
N_EDGES = 6400000
NC = 2
NS = 16
NW = NC * NS
EW = N_EDGES // NW
K = 1600
G = 80
NG = K // G
R = EW // K

L = 16
ZCHUNK = NPAD // NS

_mesh = plsc.VectorSubcoreMesh(
    core_axis_name="c", subcore_axis_name="s", num_cores=NC, num_subcores=NS
)


@functools.partial(
    pl.kernel,
    out_type=jax.ShapeDtypeStruct((NC * NPAD,), jnp.float32),
    mesh=_mesh,
    compiler_params=pltpu.CompilerParams(needs_layout_passes=False),
    scratch_types=[
        pltpu.VMEM((NPAD,), jnp.float32),
        pltpu.VMEM((K,), jnp.int32),
        pltpu.VMEM((K,), jnp.int32),
        pltpu.VMEM((K,), jnp.float32),
        pltpu.VMEM((K,), jnp.int32),
        pltpu.VMEM((K,), jnp.int32),
        pltpu.VMEM((K,), jnp.float32),
        pltpu.VMEM((NG, G), jnp.float32),
        pltpu.VMEM((NG, G), jnp.float32),
        pltpu.VMEM((NG, G), jnp.int32),
        pltpu.VMEM((NG, G), jnp.int32),
        pltpu.VMEM_SHARED((NPAD,), jnp.float32),
        pltpu.SemaphoreType.DMA,
        pltpu.SemaphoreType.DMA,
        pltpu.SemaphoreType.DMA,
    ],
)
def _sc_potential(charges_hbm, ai_hbm, aj_hbm, dist_hbm, out_hbm,
                  table_v, aiA, ajA, ddA, aiB, ajB, ddB,
                  vi_v, vj_v, si_v, sj_v, acc_sh,
                  semA, semB, sc_sem):
    c = lax.axis_index("c")
    s = lax.axis_index("s")
    w = c * NS + s

    zeros = jnp.zeros((L,), jnp.float32)

    def _zfill(t, carry):
        table_v[pl.ds(t * L, L)] = zeros
        return carry

    lax.fori_loop(0, ZCHUNK // L, _zfill, 0)
    pltpu.sync_copy(table_v.at[pl.ds(0, ZCHUNK)],
                    acc_sh.at[pl.ds(s * ZCHUNK, ZCHUNK)])

    pltpu.sync_copy(charges_hbm, table_v)
    plsc.subcore_barrier()

    ebase = w * EW

    def _fetch(r, ai_b, aj_b, dd_b, sem):
        base = ebase + r * K
        pltpu.async_copy(ai_hbm.at[pl.ds(base, K)], ai_b, sem)
        pltpu.async_copy(aj_hbm.at[pl.ds(base, K)], aj_b, sem)
        pltpu.async_copy(dist_hbm.at[pl.ds(base, K)], dd_b, sem)

    def _wait_inputs(ai_b, aj_b, dd_b, sem):
        pltpu.make_async_copy(ai_hbm.at[pl.ds(0, K)], ai_b, sem).wait()
        pltpu.make_async_copy(aj_hbm.at[pl.ds(0, K)], aj_b, sem).wait()
        pltpu.make_async_copy(dist_hbm.at[pl.ds(0, K)], dd_b, sem).wait()

    def _drain_scatter(n):
        for _ in range(n):
            pltpu.make_async_copy(vi_v.at[0], acc_sh.at[si_v.at[0]],
                                  sc_sem).wait()
            pltpu.make_async_copy(vj_v.at[0], acc_sh.at[sj_v.at[0]],
                                  sc_sem).wait()

    def _process(ai_b, aj_b, dd_b):
        def _group(g, carry):
            for u in range(G // L):
                e0 = g * G + u * L
                ii = ai_b[pl.ds(e0, L)]
                jj = aj_b[pl.ds(e0, L)]
                dd = dd_b[pl.ds(e0, L)]
                ci = plsc.load_gather(table_v, [ii])
                cj = plsc.load_gather(table_v, [jj])
                p = 0.5 / dd
                vi_v[g, pl.ds(u * L, L)] = cj * p
                vj_v[g, pl.ds(u * L, L)] = ci * p
                si_v[g, pl.ds(u * L, L)] = ii
                sj_v[g, pl.ds(u * L, L)] = jj
            pltpu.async_copy(vi_v.at[g], acc_sh.at[si_v.at[g]], sc_sem,
                             add=True)
            pltpu.async_copy(vj_v.at[g], acc_sh.at[sj_v.at[g]], sc_sem,
                             add=True)
            return carry

        lax.fori_loop(0, NG, _group, 0)
        _drain_scatter(NG)

    _fetch(0, aiA, ajA, ddA, semA)

    def _pair(t, carry):
        r = 2 * t
        _wait_inputs(aiA, ajA, ddA, semA)
        _fetch(r + 1, aiB, ajB, ddB, semB)
        _process(aiA, ajA, ddA)
        _wait_inputs(aiB, ajB, ddB, semB)

        @pl.when(r + 2 < R)
        def _():
            _fetch(r + 2, aiA, ajA, ddA, semA)

        _process(aiB, ajB, ddB)
        return carry

    lax.fori_loop(0, R // 2, _pair, 0)
    if R % 2:
        _wait_inputs(aiA, ajA, ddA, semA)
        _process(aiA, ajA, ddA)

    plsc.subcore_barrier()
    pltpu.sync_copy(acc_sh.at[pl.ds(s * ZCHUNK, ZCHUNK)],
                    table_v.at[pl.ds(0, ZCHUNK)])
    pltpu.sync_copy(table_v.at[pl.ds(0, ZCHUNK)],
                    out_hbm.at[pl.ds(c * NPAD + s * ZCHUNK, ZCHUNK)])


def kernel(charges, neighbor_indices, neighbor_distances):
    ch = jnp.pad(charges[:, 0], (0, NPAD - N_NODES))
    idx = neighbor_indices.astype(jnp.int32)
    parts = _sc_potential(ch, idx[:, 0], idx[:, 1], neighbor_distances)
    return (parts[:N_NODES] + parts[NPAD:NPAD + N_NODES]).reshape(N_NODES, 1)
